# Initial kernel scaffold; baseline (speedup 1.0000x reference)
import jax
import jax.numpy as jnp
from jax.experimental import pallas as pl


def kernel(node_input, edge_src, edge_dst, edge_attr, W_in, W_mlp0, W_mlp1, W_mlp2, W_out_scal, W_out_vec):
    raise NotImplementedError("write your pallas kernel here")


# speedup vs baseline: 22.9379x; 22.9379x over previous
import functools

import jax
import jax.numpy as jnp
from jax import lax
from jax.experimental import pallas as pl
from jax.experimental.pallas import tpu as pltpu
from jax.experimental.pallas import tpu_sc as plsc

F32 = jnp.float32


def _x_body(n_ref, w_ref, x_ref):
    xm = jnp.dot(n_ref[...], w_ref[...], preferred_element_type=F32)
    x_ref[...] = jnp.concatenate([xm, jnp.zeros_like(xm)], axis=1)


def _node_linear(node_input, w_in):
    N, C = node_input.shape
    return pl.pallas_call(
        _x_body,
        out_shape=jax.ShapeDtypeStruct((N, 2 * C), F32),
    )(node_input, w_in)


def _m_body(attr_ref, w0_ref, w1_ref, w2_ref, m_ref):
    a = attr_ref[...]
    inv = a[:, 0:1]
    h = inv * w0_ref[...]
    h = h * jax.nn.sigmoid(h)
    h = jnp.dot(h, w1_ref[...], preferred_element_type=F32)
    h = h * jax.nn.sigmoid(h)
    f = jnp.dot(h, w2_ref[...], preferred_element_type=F32)
    fv = f[:, 0:64]
    fs = f[:, 64:128]
    s = 0.25
    m0 = jnp.concatenate([fs, fv * a[:, 1:2]], axis=1) * s
    m1 = jnp.concatenate([fv * a[:, 2:3], fv * a[:, 3:4]], axis=1) * s
    m_ref[...] = jnp.stack([m0, m1], axis=0)


def _edge_multipliers(edge_attr, w0, w1, w2):
    E = edge_attr.shape[0]
    B = 2000
    grid = E // B
    return pl.pallas_call(
        _m_body,
        grid=(grid,),
        in_specs=[
            pl.BlockSpec((B, 4), lambda i: (i, 0)),
            pl.BlockSpec((1, 64), lambda i: (0, 0)),
            pl.BlockSpec((64, 64), lambda i: (0, 0)),
            pl.BlockSpec((64, 128), lambda i: (0, 0)),
        ],
        out_specs=pl.BlockSpec((2, B, 128), lambda i: (0, i, 0)),
        out_shape=jax.ShapeDtypeStruct((2, E, 128), F32),
    )(edge_attr, w0, w1, w2)


def _sc_body(E, N, x_hbm, m_hbm, src_hbm, dst_hbm, agg_hbm,
             acc, sidx, didx, xs, mv, pay, zb, sem):
    CH = 80
    per_tile = E // 16
    iters = per_tile // CH
    SLAB = 624
    ZB = 16
    tail0 = 16 * SLAB
    tail_n = N - tail0

    c = lax.axis_index("c")
    s = lax.axis_index("s")

    zero16 = jnp.zeros((16,), F32)

    def _zrow(r, carry):
        for j in range(8):
            zb[r, pl.ds(j * 16, 16)] = zero16
        return carry

    lax.fori_loop(0, ZB, _zrow, 0)
    for k in range(SLAB // ZB):
        pltpu.sync_copy(zb, acc.at[pl.ds(s * SLAB + k * ZB, ZB)])

    @pl.when(s == 0)
    def _zero_tail():
        pltpu.sync_copy(zb.at[pl.ds(0, tail_n)], acc.at[pl.ds(tail0, tail_n)])

    plsc.subcore_barrier()

    def _iter(it, carry):
        base = s * per_tile + it * CH
        pltpu.sync_copy(src_hbm.at[pl.ds(base, CH)], sidx)
        pltpu.sync_copy(dst_hbm.at[pl.ds(base, CH)], didx)
        pltpu.async_copy(x_hbm.at[sidx], xs, sem).wait()
        pltpu.sync_copy(m_hbm.at[c, pl.ds(base, CH)], mv)

        def _edge(i, ecarry):
            xv = [xs[i, pl.ds(h * 16, 16)] for h in range(4)]
            for j in range(8):
                pay[i, pl.ds(j * 16, 16)] = xv[j % 4] * mv[i, pl.ds(j * 16, 16)]
            return ecarry

        lax.fori_loop(0, CH, _edge, 0)
        pltpu.sync_copy(pay, acc.at[didx], add=True)
        return carry

    lax.fori_loop(0, iters, _iter, 0)
    plsc.subcore_barrier()

    pltpu.sync_copy(acc.at[pl.ds(s * SLAB, SLAB)],
                    agg_hbm.at[c, pl.ds(s * SLAB, SLAB)])

    @pl.when(s == 0)
    def _drain_tail():
        pltpu.sync_copy(acc.at[pl.ds(tail0, tail_n)],
                        agg_hbm.at[c, pl.ds(tail0, tail_n)])


def _sc_scatter(x, m, src, dst):
    N = x.shape[0]
    E = src.shape[0]
    CH = 80
    mesh = plsc.VectorSubcoreMesh(core_axis_name="c", subcore_axis_name="s")
    kfn = pl.kernel(
        functools.partial(_sc_body, E, N),
        out_type=jax.ShapeDtypeStruct((2, N, 128), F32),
        mesh=mesh,
        scratch_types=[
            pltpu.VMEM_SHARED((N, 128), F32),
            pltpu.VMEM((CH,), jnp.int32),
            pltpu.VMEM((CH,), jnp.int32),
            pltpu.VMEM((CH, 128), F32),
            pltpu.VMEM((CH, 128), F32),
            pltpu.VMEM((CH, 128), F32),
            pltpu.VMEM((16, 128), F32),
            pltpu.SemaphoreType.DMA,
        ],
    )
    return kfn(x, m, src, dst)


def _out_body(agg_ref, ws_ref, wv_ref, o_ref):
    g = agg_ref[0]
    a12 = agg_ref[1]
    wv = wv_ref[...]
    o_ref[0] = jnp.dot(g[:, 0:64], ws_ref[...], preferred_element_type=F32)
    o_ref[1] = jnp.dot(g[:, 64:128], wv, preferred_element_type=F32)
    o_ref[2] = jnp.dot(a12[:, 0:64], wv, preferred_element_type=F32)
    o_ref[3] = jnp.dot(a12[:, 64:128], wv, preferred_element_type=F32)


def _final_linear(agg, ws, wv):
    N = agg.shape[1]
    B = 2000
    grid = N // B
    return pl.pallas_call(
        _out_body,
        grid=(grid,),
        in_specs=[
            pl.BlockSpec((2, B, 128), lambda i: (0, i, 0)),
            pl.BlockSpec((64, 64), lambda i: (0, 0)),
            pl.BlockSpec((64, 64), lambda i: (0, 0)),
        ],
        out_specs=pl.BlockSpec((4, B, 64), lambda i: (0, i, 0)),
        out_shape=jax.ShapeDtypeStruct((4, N, 64), F32),
    )(agg, ws, wv)


def kernel(node_input, edge_src, edge_dst, edge_attr,
           W_in, W_mlp0, W_mlp1, W_mlp2, W_out_scal, W_out_vec):
    N, C = node_input.shape
    E = edge_src.shape[0]

    inv_sqrt_c = 1.0 / jnp.sqrt(jnp.float32(C))
    x = _node_linear(node_input, W_in * inv_sqrt_c)
    m = _edge_multipliers(edge_attr, W_mlp0,
                          W_mlp1 * (1.0 / jnp.sqrt(jnp.float32(64))),
                          W_mlp2 * (1.0 / jnp.sqrt(jnp.float32(64))))

    src = edge_src.astype(jnp.int32)
    dst = edge_dst.astype(jnp.int32)
    agg = _sc_scatter(x, m, src, dst)

    o4 = _final_linear(agg, W_out_scal * inv_sqrt_c, W_out_vec * inv_sqrt_c)
    out_scal = o4[0]
    out_vec = jnp.stack([o4[1], o4[2], o4[3]], axis=-1)
    return jnp.concatenate([out_scal, out_vec.reshape(N, C * 3)], axis=1)


# speedup vs baseline: 36.5396x; 1.5930x over previous
import functools

import jax
import jax.numpy as jnp
from jax import lax
from jax.experimental import pallas as pl
from jax.experimental.pallas import tpu as pltpu
from jax.experimental.pallas import tpu_sc as plsc

F32 = jnp.float32


def _x_body(n_ref, w_ref, x_ref):
    xm = jnp.dot(n_ref[...], w_ref[...], preferred_element_type=F32)
    x_ref[...] = jnp.concatenate([xm, jnp.zeros_like(xm)], axis=1)


def _node_linear(node_input, w_in):
    N, C = node_input.shape
    return pl.pallas_call(
        _x_body,
        out_shape=jax.ShapeDtypeStruct((N, 2 * C), F32),
    )(node_input, w_in)


def _m_body(attr_ref, w0_ref, w1_ref, w2_ref, m_ref):
    a = attr_ref[...]
    inv = a[:, 0:1]
    h = inv * w0_ref[...]
    h = h * jax.nn.sigmoid(h)
    h = jnp.dot(h, w1_ref[...], preferred_element_type=F32)
    h = h * jax.nn.sigmoid(h)
    f = jnp.dot(h, w2_ref[...], preferred_element_type=F32)
    fv = f[:, 0:64]
    fs = f[:, 64:128]
    s = 0.25
    m0 = jnp.concatenate([fs, fv * a[:, 1:2]], axis=1) * s
    m1 = jnp.concatenate([fv * a[:, 2:3], fv * a[:, 3:4]], axis=1) * s
    m_ref[...] = jnp.stack([m0, m1], axis=0)


def _edge_multipliers(edge_attr, w0, w1, w2):
    E = edge_attr.shape[0]
    B = 2000
    grid = E // B
    return pl.pallas_call(
        _m_body,
        grid=(grid,),
        in_specs=[
            pl.BlockSpec((B, 4), lambda i: (i, 0)),
            pl.BlockSpec((1, 64), lambda i: (0, 0)),
            pl.BlockSpec((64, 64), lambda i: (0, 0)),
            pl.BlockSpec((64, 128), lambda i: (0, 0)),
        ],
        out_specs=pl.BlockSpec((2, B, 128), lambda i: (0, i, 0)),
        out_shape=jax.ShapeDtypeStruct((2, E, 128), F32),
    )(edge_attr, w0, w1, w2)


def _sc_body(E, N, x_hbm, m_hbm, src_hbm, dst_hbm, agg_hbm,
             acc, sidx, didx, xs, mv, zb,
             sem_g, sem_m, sem_i, sem_z):
    CH = 80
    per_tile = E // 16
    iters = per_tile // CH
    SLAB = 624
    ZB = 48
    tail0 = 16 * SLAB
    tail_n = N - tail0

    c = lax.axis_index("c")
    s = lax.axis_index("s")
    tile_base = s * per_tile

    def _fire_idx(t, b):
        base = tile_base + t * CH
        pltpu.async_copy(src_hbm.at[pl.ds(base, CH)], sidx.at[b], sem_i.at[b])
        pltpu.async_copy(dst_hbm.at[pl.ds(base, CH)], didx.at[b], sem_i.at[b])

    def _wait_idx(b):
        pltpu.make_async_copy(src_hbm.at[pl.ds(0, CH)], sidx.at[b],
                              sem_i.at[b]).wait()
        pltpu.make_async_copy(dst_hbm.at[pl.ds(0, CH)], didx.at[b],
                              sem_i.at[b]).wait()

    def _fire_data(t, b):
        base = tile_base + t * CH
        pltpu.async_copy(x_hbm.at[sidx.at[b]], xs.at[b], sem_g.at[b])
        pltpu.async_copy(m_hbm.at[c, pl.ds(base, CH)], mv.at[b], sem_m.at[b])

    def _wait_data(b):
        pltpu.make_async_copy(x_hbm.at[sidx.at[b]], xs.at[b], sem_g.at[b]).wait()
        pltpu.make_async_copy(m_hbm.at[c, pl.ds(0, CH)], mv.at[b],
                              sem_m.at[b]).wait()

    pltpu.sync_copy(src_hbm.at[pl.ds(tile_base, CH)], sidx.at[0])
    pltpu.sync_copy(dst_hbm.at[pl.ds(tile_base, CH)], didx.at[0])
    _fire_data(0, 0)
    _fire_idx(1, 1)

    zero16 = jnp.zeros((16,), F32)

    def _zrow(r, carry):
        for j in range(8):
            zb[r, pl.ds(j * 16, 16)] = zero16
        return carry

    lax.fori_loop(0, ZB, _zrow, 0)
    for k in range(SLAB // ZB):
        pltpu.async_copy(zb, acc.at[pl.ds(s * SLAB + k * ZB, ZB)], sem_z)
    for k in range(SLAB // ZB):
        pltpu.make_async_copy(zb, acc.at[pl.ds(0, ZB)], sem_z).wait()

    @pl.when(s == 0)
    def _zero_tail():
        pltpu.sync_copy(zb.at[pl.ds(0, tail_n)], acc.at[pl.ds(tail0, tail_n)])

    plsc.subcore_barrier()

    def _compute(b):
        def _edge(i, ecarry):
            xv = [xs[b, i, pl.ds(h * 16, 16)] for h in range(4)]
            for j in range(8):
                mv[b, i, pl.ds(j * 16, 16)] = (
                    xv[j % 4] * mv[b, i, pl.ds(j * 16, 16)])
            return ecarry

        lax.fori_loop(0, CH, _edge, 0)

    def _stage(t, b):
        _wait_idx(1 - b)
        _fire_data(t + 1, 1 - b)
        _wait_data(b)
        _compute(b)
        pltpu.sync_copy(mv.at[b], acc.at[didx.at[b]], add=True)

        @pl.when(t + 2 < iters)
        def _prefetch_idx():
            _fire_idx(t + 2, b)

    def _pair(k, carry):
        _stage(2 * k, 0)
        _stage(2 * k + 1, 1)
        return carry

    lax.fori_loop(0, (iters - 1) // 2, _pair, 0)
    _wait_data(0)
    _compute(0)
    pltpu.sync_copy(mv.at[0], acc.at[didx.at[0]], add=True)
    plsc.subcore_barrier()

    pltpu.sync_copy(acc.at[pl.ds(s * SLAB, SLAB)],
                    agg_hbm.at[c, pl.ds(s * SLAB, SLAB)])

    @pl.when(s == 0)
    def _drain_tail():
        pltpu.sync_copy(acc.at[pl.ds(tail0, tail_n)],
                        agg_hbm.at[c, pl.ds(tail0, tail_n)])


def _sc_scatter(x, m, src, dst):
    N = x.shape[0]
    E = src.shape[0]
    CH = 80
    mesh = plsc.VectorSubcoreMesh(core_axis_name="c", subcore_axis_name="s")
    kfn = pl.kernel(
        functools.partial(_sc_body, E, N),
        out_type=jax.ShapeDtypeStruct((2, N, 128), F32),
        mesh=mesh,
        scratch_types=[
            pltpu.VMEM_SHARED((N, 128), F32),
            pltpu.VMEM((2, CH), jnp.int32),
            pltpu.VMEM((2, CH), jnp.int32),
            pltpu.VMEM((2, CH, 128), F32),
            pltpu.VMEM((2, CH, 128), F32),
            pltpu.VMEM((48, 128), F32),
            pltpu.SemaphoreType.DMA((2,)),
            pltpu.SemaphoreType.DMA((2,)),
            pltpu.SemaphoreType.DMA((2,)),
            pltpu.SemaphoreType.DMA,
        ],
    )
    return kfn(x, m, src, dst)


def _out_body(agg_ref, ws_ref, wv_ref, o_ref):
    g = agg_ref[0]
    a12 = agg_ref[1]
    wv = wv_ref[...]
    o_ref[0] = jnp.dot(g[:, 0:64], ws_ref[...], preferred_element_type=F32)
    o_ref[1] = jnp.dot(g[:, 64:128], wv, preferred_element_type=F32)
    o_ref[2] = jnp.dot(a12[:, 0:64], wv, preferred_element_type=F32)
    o_ref[3] = jnp.dot(a12[:, 64:128], wv, preferred_element_type=F32)


def _final_linear(agg, ws, wv):
    N = agg.shape[1]
    B = 2000
    grid = N // B
    return pl.pallas_call(
        _out_body,
        grid=(grid,),
        in_specs=[
            pl.BlockSpec((2, B, 128), lambda i: (0, i, 0)),
            pl.BlockSpec((64, 64), lambda i: (0, 0)),
            pl.BlockSpec((64, 64), lambda i: (0, 0)),
        ],
        out_specs=pl.BlockSpec((4, B, 64), lambda i: (0, i, 0)),
        out_shape=jax.ShapeDtypeStruct((4, N, 64), F32),
    )(agg, ws, wv)


def kernel(node_input, edge_src, edge_dst, edge_attr,
           W_in, W_mlp0, W_mlp1, W_mlp2, W_out_scal, W_out_vec):
    N, C = node_input.shape
    E = edge_src.shape[0]

    inv_sqrt_c = 1.0 / jnp.sqrt(jnp.float32(C))
    x = _node_linear(node_input, W_in * inv_sqrt_c)
    m = _edge_multipliers(edge_attr, W_mlp0,
                          W_mlp1 * (1.0 / jnp.sqrt(jnp.float32(64))),
                          W_mlp2 * (1.0 / jnp.sqrt(jnp.float32(64))))

    src = edge_src.astype(jnp.int32)
    dst = edge_dst.astype(jnp.int32)
    agg = _sc_scatter(x, m, src, dst)

    o4 = _final_linear(agg, W_out_scal * inv_sqrt_c, W_out_vec * inv_sqrt_c)
    out_scal = o4[0]
    out_vec = jnp.stack([o4[1], o4[2], o4[3]], axis=-1)
    return jnp.concatenate([out_scal, out_vec.reshape(N, C * 3)], axis=1)
